# baseline (speedup 1.0000x reference)
import functools

import jax
import jax.numpy as jnp
import numpy as np
from jax import lax
from jax.experimental import pallas as pl
from jax.experimental.pallas import tpu as pltpu

B = 1024
L = 20
DIM = 100
NUM_TOTAL = 100000
ALPHA = 0.2

BB = 32
BBL = BB * L
NBLK = B // BB
NEG = -9e15

NW = 32
DPAD = 128
SEG = B * L
ROWS_TOTAL = 3 * SEG
IDX_ROWS_PER_W = B // NW
SEG_PER_W = IDX_ROWS_PER_W * L


def _make_sc_gather():
    from jax.experimental.pallas import tpu_sc as plsc

    mesh = plsc.VectorSubcoreMesh(core_axis_name="c", subcore_axis_name="s")

    @functools.partial(
        pl.kernel,
        mesh=mesh,
        out_type=jax.ShapeDtypeStruct((ROWS_TOTAL, DPAD), jnp.float32),
        scratch_types=[
            pltpu.VMEM((IDX_ROWS_PER_W, L), jnp.int32),
            pltpu.VMEM((SEG_PER_W, DPAD), jnp.float32),
            pltpu.SemaphoreType.DMA,
        ],
    )
    def gather_kernel(table_hbm, i0_hbm, i1_hbm, i2_hbm, out_hbm,
                      idx_v, rows_v, sem):
        wid = lax.axis_index("s") * 2 + lax.axis_index("c")
        ibase = wid * IDX_ROWS_PER_W

        for seg, idx_hbm in enumerate((i0_hbm, i1_hbm, i2_hbm)):
            pltpu.sync_copy(idx_hbm.at[pl.ds(ibase, IDX_ROWS_PER_W)], idx_v)
            copies = []
            for r in range(IDX_ROWS_PER_W):
                copies.append(pltpu.async_copy(
                    table_hbm.at[idx_v.at[r]],
                    rows_v.at[pl.ds(r * L, L)], sem))
            for cp in copies:
                cp.wait()
            pltpu.sync_copy(
                rows_v, out_hbm.at[pl.ds(seg * SEG + wid * SEG_PER_W,
                                         SEG_PER_W)])

    return gather_kernel


PAD_ROWS = 2000


def _pad_body(src_ref, dst_ref):
    blk = jnp.concatenate([
        src_ref[...],
        jnp.ones((PAD_ROWS, 1), jnp.float32),
        jnp.zeros((PAD_ROWS, DPAD - DIM - 1), jnp.float32),
    ], axis=1)
    dst_ref[...] = blk


def _pad_table(emb):
    return pl.pallas_call(
        _pad_body,
        grid=(NUM_TOTAL // PAD_ROWS,),
        in_specs=[pl.BlockSpec((PAD_ROWS, DIM), lambda i: (i, 0))],
        out_specs=pl.BlockSpec((PAD_ROWS, DPAD), lambda i: (i, 0)),
        out_shape=jax.ShapeDtypeStruct((NUM_TOTAL, DPAD), jnp.float32),
    )(emb)


def _local_agg_block(Xp, adjt, a_t, fill):
    X = Xp[:, :DIM]
    XT = X.T
    acc = jnp.zeros((BBL, BBL), jnp.float32)
    for k in range(4):
        Ek = jnp.dot(X * a_t[k][None, :], XT,
                     preferred_element_type=jnp.float32)
        acc = acc + jnp.where(adjt == k + 1, Ek, 0.0)
    e = jnp.maximum(acc, ALPHA * acc)
    logits = jnp.where(adjt > 0, e, fill)
    m = jnp.max(logits, axis=1, keepdims=True)
    p = jnp.exp(logits - m)
    alpha = p / jnp.sum(p, axis=1, keepdims=True)
    return jnp.dot(alpha, X, preferred_element_type=jnp.float32)


def _tc_body(h1_ref, adjt_ref, h2_ref, ain_ref, aout_ref, hm_ref, tadjt_ref,
             la1_ref, mix_ref,
             weiT_ref, bei_ref, weoT_ref, beo_ref,
             winT_r, winT_i, winT_n, woutT_r, woutT_i, woutT_n,
             whhT_r, whhT_i, whhT_n,
             bih_r, bih_i, bih_n, bhh_r, bhh_i, bhh_n,
             biah_ref, boah_ref,
             fill_ref,
             o1_ref, o2_ref, om_ref):
    fill = fill_ref[...]

    o1_ref[...] = _local_agg_block(h1_ref[...], adjt_ref[...], la1_ref[...],
                                   fill)

    om_ref[...] = _local_agg_block(hm_ref[...], tadjt_ref[...], mix_ref[...],
                                   fill)

    X2 = h2_ref[...][:, :DIM]
    dot = lambda a, b: jnp.dot(a, b, preferred_element_type=jnp.float32)
    Vi = dot(X2, weiT_ref[...]) + bei_ref[...]
    Vo = dot(X2, weoT_ref[...]) + beo_ref[...]
    input_in = dot(ain_ref[...], Vi) + biah_ref[...]
    input_out = dot(aout_ref[...], Vo) + boah_ref[...]

    def gate(winT, woutT, whhT, bih, bhh):
        return (dot(input_in, winT[...]) + dot(input_out, woutT[...])
                + bih[...] + dot(X2, whhT[...]) + bhh[...])

    resetgate = jax.nn.sigmoid(gate(winT_r, woutT_r, whhT_r, bih_r, bhh_r))
    inputgate = jax.nn.sigmoid(gate(winT_i, woutT_i, whhT_i, bih_i, bhh_i))
    gin = (dot(input_in, winT_n[...]) + dot(input_out, woutT_n[...])
           + bih_n[...])
    ghn = dot(X2, whhT_n[...]) + bhh_n[...]
    newgate = jnp.tanh(gin + resetgate * ghn)
    o2_ref[...] = newgate + inputgate * (newgate - X2)


_FILL = None


def _fill_const():
    global _FILL
    if _FILL is None:
        r = np.arange(BBL)[:, None] // L
        c = np.arange(BBL)[None, :] // L
        _FILL = jnp.asarray(np.where(r == c, NEG, -np.inf).astype(np.float32))
    return _FILL


def _expand_blockdiag(x, dtype):
    xp = x.reshape(NBLK, BB, L, L).astype(dtype)
    eye = jnp.eye(BB, dtype=dtype)
    big = xp[:, :, :, None, :] * eye[None, :, None, :, None]
    return big.reshape(B * L, BBL)


def _tc_call(rows, adjt, ainb, aoutb, tadjt, weights, interpret=False):
    if isinstance(rows, tuple):
        h1r, h2r, hmr = rows
        hwid = h1r.shape[1]
        s2 = pl.BlockSpec((BBL, hwid), lambda i: (i, 0))
        s3 = pl.BlockSpec((BBL, hwid), lambda i: (i, 0))
    else:
        h1r = h2r = hmr = rows
        hwid = rows.shape[1]
        off = SEG // BBL
        s2 = pl.BlockSpec((BBL, hwid), lambda i: (i + off, 0))
        s3 = pl.BlockSpec((BBL, hwid), lambda i: (i + 2 * off, 0))
    s1 = pl.BlockSpec((BBL, hwid), lambda i: (i, 0))

    big = pl.BlockSpec((BBL, BBL), lambda i: (i, 0))
    full = lambda a: pl.BlockSpec(a.shape, lambda i: (0,) * a.ndim)

    fill = _fill_const()
    in_specs = [s1, big, s2, big, big, s3, big] + \
        [full(w) for w in weights] + [full(fill)]

    out_specs = [pl.BlockSpec((BBL, DIM), lambda i: (i, 0))] * 3
    out_shape = [jax.ShapeDtypeStruct((B * L, DIM), jnp.float32)] * 3

    return pl.pallas_call(
        _tc_body,
        grid=(NBLK,),
        in_specs=in_specs,
        out_specs=out_specs,
        out_shape=out_shape,
        interpret=interpret,
    )(h1r, adjt, h2r, ainb, aoutb, hmr, tadjt, *weights, fill)


def _prep_weights(la1_a, mix_a, Wei, bei, Weo, beo, w_ih, w_hh, b_ih, b_hh,
                  b_iah, b_oah):
    w_g = [w_ih[g * DIM:(g + 1) * DIM] for g in range(3)]
    row = lambda v: v.reshape(1, -1)
    return [
        la1_a.T, mix_a.T,
        Wei.T, row(bei), Weo.T, row(beo),
        w_g[0][:, :DIM].T, w_g[1][:, :DIM].T, w_g[2][:, :DIM].T,
        w_g[0][:, DIM:].T, w_g[1][:, DIM:].T, w_g[2][:, DIM:].T,
        w_hh[:DIM].T, w_hh[DIM:2 * DIM].T, w_hh[2 * DIM:].T,
        row(b_ih[:DIM]), row(b_ih[DIM:2 * DIM]), row(b_ih[2 * DIM:]),
        row(b_hh[:DIM]), row(b_hh[DIM:2 * DIM]), row(b_hh[2 * DIM:]),
        row(b_iah), row(b_oah),
    ]


def kernel(inputs, adj, mask_item, item, items_ID, adj_ID, total_items,
           total_adj, embedding, la1_a, mix_a, Wei, bei, Weo, beo,
           w_ih, w_hh, b_ih, b_hh, b_iah, b_oah):
    emb_p = _pad_table(embedding)
    rows = _make_sc_gather()(
        emb_p, inputs.astype(jnp.int32), items_ID.astype(jnp.int32),
        total_items.astype(jnp.int32))

    adjt = _expand_blockdiag(adj.reshape(B * L, L), jnp.int32)
    ainb = _expand_blockdiag(adj_ID[:, :, :L].reshape(B * L, L), jnp.float32)
    aoutb = _expand_blockdiag(adj_ID[:, :, L:].reshape(B * L, L), jnp.float32)
    tadjt = _expand_blockdiag(total_adj.reshape(B * L, L), jnp.int32)

    weights = _prep_weights(la1_a, mix_a, Wei, bei, Weo, beo, w_ih, w_hh,
                            b_ih, b_hh, b_iah, b_oah)

    o1, o2, om = _tc_call(rows, adjt, ainb, aoutb, tadjt, weights)

    shp = (B, L, DIM)
    return (o1.reshape(shp), o2.reshape(shp), om.reshape(shp))


# speedup vs baseline: 4.0080x; 4.0080x over previous
import functools

import jax
import jax.numpy as jnp
import numpy as np
from jax import lax
from jax.experimental import pallas as pl
from jax.experimental.pallas import tpu as pltpu

B = 1024
L = 20
DIM = 100
NUM_TOTAL = 100000
ALPHA = 0.2

BB = 8
BBL = BB * L
NBLK = B // BB
NEG = -9e15

NW = 32
DPAD = 128
SEG = B * L
ROWS_TOTAL = 3 * SEG
IDX_ROWS_PER_W = B // NW
SEG_PER_W = IDX_ROWS_PER_W * L


def _make_sc_gather():
    from jax.experimental.pallas import tpu_sc as plsc

    mesh = plsc.VectorSubcoreMesh(core_axis_name="c", subcore_axis_name="s")

    @functools.partial(
        pl.kernel,
        mesh=mesh,
        out_type=jax.ShapeDtypeStruct((ROWS_TOTAL, DPAD), jnp.float32),
        scratch_types=[
            pltpu.VMEM((IDX_ROWS_PER_W, L), jnp.int32),
            pltpu.VMEM((SEG_PER_W, DPAD), jnp.float32),
            pltpu.SemaphoreType.DMA,
        ],
    )
    def gather_kernel(table_hbm, i0_hbm, i1_hbm, i2_hbm, out_hbm,
                      idx_v, rows_v, sem):
        wid = lax.axis_index("s") * 2 + lax.axis_index("c")
        ibase = wid * IDX_ROWS_PER_W

        for seg, idx_hbm in enumerate((i0_hbm, i1_hbm, i2_hbm)):
            pltpu.sync_copy(idx_hbm.at[pl.ds(ibase, IDX_ROWS_PER_W)], idx_v)
            copies = []
            for r in range(IDX_ROWS_PER_W):
                copies.append(pltpu.async_copy(
                    table_hbm.at[idx_v.at[r]],
                    rows_v.at[pl.ds(r * L, L)], sem))
            for cp in copies:
                cp.wait()
            pltpu.sync_copy(
                rows_v, out_hbm.at[pl.ds(seg * SEG + wid * SEG_PER_W,
                                         SEG_PER_W)])

    return gather_kernel


PAD_ROWS = 2000


def _pad_body(src_ref, dst_ref):
    blk = jnp.concatenate([
        src_ref[...],
        jnp.ones((PAD_ROWS, 1), jnp.float32),
        jnp.zeros((PAD_ROWS, DPAD - DIM - 1), jnp.float32),
    ], axis=1)
    dst_ref[...] = blk


def _pad_table(emb):
    return pl.pallas_call(
        _pad_body,
        grid=(NUM_TOTAL // PAD_ROWS,),
        in_specs=[pl.BlockSpec((PAD_ROWS, DIM), lambda i: (i, 0))],
        out_specs=pl.BlockSpec((PAD_ROWS, DPAD), lambda i: (i, 0)),
        out_shape=jax.ShapeDtypeStruct((NUM_TOTAL, DPAD), jnp.float32),
    )(emb)


def _local_agg_block(Xp, adj_small, a_t, fill, tmat):
    X = Xp[:, :DIM]
    XT = X.T
    adj_t = jnp.dot(adj_small.astype(jnp.float32), tmat,
                    preferred_element_type=jnp.float32)
    acc = jnp.zeros((BBL, BBL), jnp.float32)
    for k in range(4):
        Ek = jnp.dot(X * a_t[k][None, :], XT,
                     preferred_element_type=jnp.float32)
        acc = acc + jnp.where(adj_t == float(k + 1), Ek, 0.0)
    e = jnp.maximum(acc, ALPHA * acc)
    logits = jnp.where((adj_t > 0) & (fill > -jnp.inf), e, fill)
    m = jnp.max(logits, axis=1, keepdims=True)
    p = jnp.exp(logits - m)
    alpha = p / jnp.sum(p, axis=1, keepdims=True)
    return jnp.dot(alpha, X, preferred_element_type=jnp.float32)


def _tc_body(h1_ref, adjt_ref, h2_ref, ain_ref, aout_ref, hm_ref, tadjt_ref,
             la1_ref, mix_ref,
             weiT_ref, bei_ref, weoT_ref, beo_ref,
             winT_r, winT_i, winT_n, woutT_r, woutT_i, woutT_n,
             whhT_r, whhT_i, whhT_n,
             bih_r, bih_i, bih_n, bhh_r, bhh_i, bhh_n,
             biah_ref, boah_ref,
             fill_ref, tmat_ref,
             o1_ref, o2_ref, om_ref):
    fill = fill_ref[...]
    tmat = tmat_ref[...]

    o1_ref[...] = _local_agg_block(h1_ref[...], adjt_ref[...], la1_ref[...],
                                   fill, tmat)

    om_ref[...] = _local_agg_block(hm_ref[...], tadjt_ref[...], mix_ref[...],
                                   fill, tmat)

    X2 = h2_ref[...][:, :DIM]
    samef = (fill > -jnp.inf).astype(jnp.float32)
    dot = lambda a, b: jnp.dot(a, b, preferred_element_type=jnp.float32)
    Ain = dot(ain_ref[...], tmat) * samef
    Aout = dot(aout_ref[...], tmat) * samef
    Vi = dot(X2, weiT_ref[...]) + bei_ref[...]
    Vo = dot(X2, weoT_ref[...]) + beo_ref[...]
    input_in = dot(Ain, Vi) + biah_ref[...]
    input_out = dot(Aout, Vo) + boah_ref[...]

    def gate(winT, woutT, whhT, bih, bhh):
        return (dot(input_in, winT[...]) + dot(input_out, woutT[...])
                + bih[...] + dot(X2, whhT[...]) + bhh[...])

    resetgate = jax.nn.sigmoid(gate(winT_r, woutT_r, whhT_r, bih_r, bhh_r))
    inputgate = jax.nn.sigmoid(gate(winT_i, woutT_i, whhT_i, bih_i, bhh_i))
    gin = (dot(input_in, winT_n[...]) + dot(input_out, woutT_n[...])
           + bih_n[...])
    ghn = dot(X2, whhT_n[...]) + bhh_n[...]
    newgate = jnp.tanh(gin + resetgate * ghn)
    o2_ref[...] = newgate + inputgate * (newgate - X2)


_FILL = None
_TMAT = None


def _fill_const():
    global _FILL
    if _FILL is None:
        r = np.arange(BBL)[:, None] // L
        c = np.arange(BBL)[None, :] // L
        _FILL = jnp.asarray(np.where(r == c, NEG, -np.inf).astype(np.float32))
    return _FILL


def _tmat_const():
    global _TMAT
    if _TMAT is None:
        _TMAT = jnp.asarray(np.tile(np.eye(L, dtype=np.float32), (1, BB)))
    return _TMAT


def _expand_blockdiag(x, dtype):
    xp = x.reshape(NBLK, BB, L, L).astype(dtype)
    eye = jnp.eye(BB, dtype=dtype)
    big = xp[:, :, :, None, :] * eye[None, :, None, :, None]
    return big.reshape(B * L, BBL)


def _tc_call(rows, adjt, ainb, aoutb, tadjt, weights, interpret=False):
    if isinstance(rows, tuple):
        h1r, h2r, hmr = rows
        hwid = h1r.shape[1]
        s2 = pl.BlockSpec((BBL, hwid), lambda i: (i, 0))
        s3 = pl.BlockSpec((BBL, hwid), lambda i: (i, 0))
    else:
        h1r = h2r = hmr = rows
        hwid = rows.shape[1]
        off = SEG // BBL
        s2 = pl.BlockSpec((BBL, hwid), lambda i: (i + off, 0))
        s3 = pl.BlockSpec((BBL, hwid), lambda i: (i + 2 * off, 0))
    s1 = pl.BlockSpec((BBL, hwid), lambda i: (i, 0))

    small = pl.BlockSpec((BBL, L), lambda i: (i, 0))
    full = lambda a: pl.BlockSpec(a.shape, lambda i: (0,) * a.ndim)

    fill = _fill_const()
    tmat = _tmat_const()
    in_specs = [s1, small, s2, small, small, s3, small] + \
        [full(w) for w in weights] + [full(fill), full(tmat)]

    out_specs = [pl.BlockSpec((BBL, DIM), lambda i: (i, 0))] * 3
    out_shape = [jax.ShapeDtypeStruct((B * L, DIM), jnp.float32)] * 3

    return pl.pallas_call(
        _tc_body,
        grid=(NBLK,),
        in_specs=in_specs,
        out_specs=out_specs,
        out_shape=out_shape,
        interpret=interpret,
    )(h1r, adjt, h2r, ainb, aoutb, hmr, tadjt, *weights, fill, tmat)


def _prep_weights(la1_a, mix_a, Wei, bei, Weo, beo, w_ih, w_hh, b_ih, b_hh,
                  b_iah, b_oah):
    w_g = [w_ih[g * DIM:(g + 1) * DIM] for g in range(3)]
    row = lambda v: v.reshape(1, -1)
    return [
        la1_a.T, mix_a.T,
        Wei.T, row(bei), Weo.T, row(beo),
        w_g[0][:, :DIM].T, w_g[1][:, :DIM].T, w_g[2][:, :DIM].T,
        w_g[0][:, DIM:].T, w_g[1][:, DIM:].T, w_g[2][:, DIM:].T,
        w_hh[:DIM].T, w_hh[DIM:2 * DIM].T, w_hh[2 * DIM:].T,
        row(b_ih[:DIM]), row(b_ih[DIM:2 * DIM]), row(b_ih[2 * DIM:]),
        row(b_hh[:DIM]), row(b_hh[DIM:2 * DIM]), row(b_hh[2 * DIM:]),
        row(b_iah), row(b_oah),
    ]


def kernel(inputs, adj, mask_item, item, items_ID, adj_ID, total_items,
           total_adj, embedding, la1_a, mix_a, Wei, bei, Weo, beo,
           w_ih, w_hh, b_ih, b_hh, b_iah, b_oah):
    emb_p = _pad_table(embedding)
    rows = _make_sc_gather()(
        emb_p, inputs.astype(jnp.int32), items_ID.astype(jnp.int32),
        total_items.astype(jnp.int32))

    adjt = adj.reshape(B * L, L)
    ainb = adj_ID[:, :, :L].reshape(B * L, L)
    aoutb = adj_ID[:, :, L:].reshape(B * L, L)
    tadjt = total_adj.reshape(B * L, L)

    weights = _prep_weights(la1_a, mix_a, Wei, bei, Weo, beo, w_ih, w_hh,
                            b_ih, b_hh, b_iah, b_oah)

    o1, o2, om = _tc_call(rows, adjt, ainb, aoutb, tadjt, weights)

    shp = (B, L, DIM)
    return (o1.reshape(shp), o2.reshape(shp), om.reshape(shp))


# speedup vs baseline: 4.1741x; 1.0414x over previous
import functools

import jax
import jax.numpy as jnp
import numpy as np
from jax import lax
from jax.experimental import pallas as pl
from jax.experimental.pallas import tpu as pltpu

B = 1024
L = 20
DIM = 100
NUM_TOTAL = 100000
ALPHA = 0.2

BB = 8
BBL = BB * L
NBLK = B // BB
NEG = -9e15

NW = 32
DPAD = 128
SEG = B * L
ROWS_TOTAL = 3 * SEG
IDX_ROWS_PER_W = B // NW
SEG_PER_W = IDX_ROWS_PER_W * L


def _make_sc_gather():
    from jax.experimental.pallas import tpu_sc as plsc

    mesh = plsc.VectorSubcoreMesh(core_axis_name="c", subcore_axis_name="s")

    @functools.partial(
        pl.kernel,
        mesh=mesh,
        out_type=jax.ShapeDtypeStruct((ROWS_TOTAL, DPAD), jnp.float32),
        compiler_params=pltpu.CompilerParams(use_tc_tiling_on_sc=True),
        scratch_types=[
            pltpu.VMEM((IDX_ROWS_PER_W, L), jnp.int32),
            pltpu.VMEM((SEG_PER_W, DPAD), jnp.float32),
            pltpu.SemaphoreType.DMA,
        ],
    )
    def gather_kernel(table_hbm, i0_hbm, i1_hbm, i2_hbm, out_hbm,
                      idx_v, rows_v, sem):
        wid = lax.axis_index("s") * 2 + lax.axis_index("c")
        ibase = wid * IDX_ROWS_PER_W

        for seg, idx_hbm in enumerate((i0_hbm, i1_hbm, i2_hbm)):
            pltpu.sync_copy(idx_hbm.at[pl.ds(ibase, IDX_ROWS_PER_W)], idx_v)
            copies = []
            for r in range(IDX_ROWS_PER_W):
                copies.append(pltpu.async_copy(
                    table_hbm.at[idx_v.at[r]],
                    rows_v.at[pl.ds(r * L, L)], sem))
            for cp in copies:
                cp.wait()
            pltpu.sync_copy(
                rows_v, out_hbm.at[pl.ds(seg * SEG + wid * SEG_PER_W,
                                         SEG_PER_W)])

    return gather_kernel


PAD_ROWS = 10000


def _pad_body(src_ref, dst_ref):
    blk = jnp.concatenate([
        src_ref[...],
        jnp.ones((PAD_ROWS, 1), jnp.float32),
        jnp.zeros((PAD_ROWS, DPAD - DIM - 1), jnp.float32),
    ], axis=1)
    dst_ref[...] = blk


def _pad_table(emb):
    return pl.pallas_call(
        _pad_body,
        grid=(NUM_TOTAL // PAD_ROWS,),
        in_specs=[pl.BlockSpec((PAD_ROWS, DIM), lambda i: (i, 0))],
        out_specs=pl.BlockSpec((PAD_ROWS, DPAD), lambda i: (i, 0)),
        out_shape=jax.ShapeDtypeStruct((NUM_TOTAL, DPAD), jnp.float32),
    )(emb)


def _local_agg_block(Xp, adj_small, a_t, fill, tmat):
    X = Xp[:, :DIM]
    XT = X.T
    adj_t = jnp.dot(adj_small.astype(jnp.float32), tmat,
                    preferred_element_type=jnp.float32)
    acc = jnp.zeros((BBL, BBL), jnp.float32)
    for k in range(4):
        Ek = jnp.dot(X * a_t[k][None, :], XT,
                     preferred_element_type=jnp.float32)
        acc = acc + jnp.where(adj_t == float(k + 1), Ek, 0.0)
    e = jnp.maximum(acc, ALPHA * acc)
    logits = jnp.where((adj_t > 0) & (fill > -jnp.inf), e, fill)
    m = jnp.max(logits, axis=1, keepdims=True)
    p = jnp.exp(logits - m)
    alpha = p / jnp.sum(p, axis=1, keepdims=True)
    return jnp.dot(alpha, X, preferred_element_type=jnp.float32)


def _tc_body(h1_ref, adjt_ref, h2_ref, ain_ref, aout_ref, hm_ref, tadjt_ref,
             la1_ref, mix_ref,
             weiT_ref, bei_ref, weoT_ref, beo_ref,
             winT_r, winT_i, winT_n, woutT_r, woutT_i, woutT_n,
             whhT_r, whhT_i, whhT_n,
             bih_r, bih_i, bih_n, bhh_r, bhh_i, bhh_n,
             biah_ref, boah_ref,
             fill_ref, tmat_ref,
             o1_ref, o2_ref, om_ref):
    fill = fill_ref[...]
    tmat = tmat_ref[...]

    o1_ref[...] = _local_agg_block(h1_ref[...], adjt_ref[...], la1_ref[...],
                                   fill, tmat)

    om_ref[...] = _local_agg_block(hm_ref[...], tadjt_ref[...], mix_ref[...],
                                   fill, tmat)

    X2 = h2_ref[...][:, :DIM]
    samef = (fill > -jnp.inf).astype(jnp.float32)
    dot = lambda a, b: jnp.dot(a, b, preferred_element_type=jnp.float32)
    Ain = dot(ain_ref[...], tmat) * samef
    Aout = dot(aout_ref[...], tmat) * samef
    Vi = dot(X2, weiT_ref[...]) + bei_ref[...]
    Vo = dot(X2, weoT_ref[...]) + beo_ref[...]
    input_in = dot(Ain, Vi) + biah_ref[...]
    input_out = dot(Aout, Vo) + boah_ref[...]

    def gate(winT, woutT, whhT, bih, bhh):
        return (dot(input_in, winT[...]) + dot(input_out, woutT[...])
                + bih[...] + dot(X2, whhT[...]) + bhh[...])

    resetgate = jax.nn.sigmoid(gate(winT_r, woutT_r, whhT_r, bih_r, bhh_r))
    inputgate = jax.nn.sigmoid(gate(winT_i, woutT_i, whhT_i, bih_i, bhh_i))
    gin = (dot(input_in, winT_n[...]) + dot(input_out, woutT_n[...])
           + bih_n[...])
    ghn = dot(X2, whhT_n[...]) + bhh_n[...]
    newgate = jnp.tanh(gin + resetgate * ghn)
    o2_ref[...] = newgate + inputgate * (newgate - X2)


_FILL = None
_TMAT = None


def _fill_const():
    global _FILL
    if _FILL is None:
        r = np.arange(BBL)[:, None] // L
        c = np.arange(BBL)[None, :] // L
        _FILL = jnp.asarray(np.where(r == c, NEG, -np.inf).astype(np.float32))
    return _FILL


def _tmat_const():
    global _TMAT
    if _TMAT is None:
        _TMAT = jnp.asarray(np.tile(np.eye(L, dtype=np.float32), (1, BB)))
    return _TMAT


def _expand_blockdiag(x, dtype):
    xp = x.reshape(NBLK, BB, L, L).astype(dtype)
    eye = jnp.eye(BB, dtype=dtype)
    big = xp[:, :, :, None, :] * eye[None, :, None, :, None]
    return big.reshape(B * L, BBL)


def _tc_call(rows, adjt, ainb, aoutb, tadjt, weights, interpret=False):
    if isinstance(rows, tuple):
        h1r, h2r, hmr = rows
        hwid = h1r.shape[1]
        s2 = pl.BlockSpec((BBL, hwid), lambda i: (i, 0))
        s3 = pl.BlockSpec((BBL, hwid), lambda i: (i, 0))
    else:
        h1r = h2r = hmr = rows
        hwid = rows.shape[1]
        off = SEG // BBL
        s2 = pl.BlockSpec((BBL, hwid), lambda i: (i + off, 0))
        s3 = pl.BlockSpec((BBL, hwid), lambda i: (i + 2 * off, 0))
    s1 = pl.BlockSpec((BBL, hwid), lambda i: (i, 0))

    small = pl.BlockSpec((BBL, L), lambda i: (i, 0))
    full = lambda a: pl.BlockSpec(a.shape, lambda i: (0,) * a.ndim)

    fill = _fill_const()
    tmat = _tmat_const()
    in_specs = [s1, small, s2, small, small, s3, small] + \
        [full(w) for w in weights] + [full(fill), full(tmat)]

    out_specs = [pl.BlockSpec((BBL, DIM), lambda i: (i, 0))] * 3
    out_shape = [jax.ShapeDtypeStruct((B * L, DIM), jnp.float32)] * 3

    return pl.pallas_call(
        _tc_body,
        grid=(NBLK,),
        in_specs=in_specs,
        out_specs=out_specs,
        out_shape=out_shape,
        interpret=interpret,
    )(h1r, adjt, h2r, ainb, aoutb, hmr, tadjt, *weights, fill, tmat)


def _prep_weights(la1_a, mix_a, Wei, bei, Weo, beo, w_ih, w_hh, b_ih, b_hh,
                  b_iah, b_oah):
    w_g = [w_ih[g * DIM:(g + 1) * DIM] for g in range(3)]
    row = lambda v: v.reshape(1, -1)
    return [
        la1_a.T, mix_a.T,
        Wei.T, row(bei), Weo.T, row(beo),
        w_g[0][:, :DIM].T, w_g[1][:, :DIM].T, w_g[2][:, :DIM].T,
        w_g[0][:, DIM:].T, w_g[1][:, DIM:].T, w_g[2][:, DIM:].T,
        w_hh[:DIM].T, w_hh[DIM:2 * DIM].T, w_hh[2 * DIM:].T,
        row(b_ih[:DIM]), row(b_ih[DIM:2 * DIM]), row(b_ih[2 * DIM:]),
        row(b_hh[:DIM]), row(b_hh[DIM:2 * DIM]), row(b_hh[2 * DIM:]),
        row(b_iah), row(b_oah),
    ]


def kernel(inputs, adj, mask_item, item, items_ID, adj_ID, total_items,
           total_adj, embedding, la1_a, mix_a, Wei, bei, Weo, beo,
           w_ih, w_hh, b_ih, b_hh, b_iah, b_oah):
    emb_p = _pad_table(embedding)
    rows = _make_sc_gather()(
        emb_p, inputs.astype(jnp.int32), items_ID.astype(jnp.int32),
        total_items.astype(jnp.int32))

    adjt = adj.reshape(B * L, L)
    ainb = adj_ID[:, :, :L].reshape(B * L, L)
    aoutb = adj_ID[:, :, L:].reshape(B * L, L)
    tadjt = total_adj.reshape(B * L, L)

    weights = _prep_weights(la1_a, mix_a, Wei, bei, Weo, beo, w_ih, w_hh,
                            b_ih, b_hh, b_iah, b_oah)

    o1, o2, om = _tc_call(rows, adjt, ainb, aoutb, tadjt, weights)

    shp = (B, L, DIM)
    return (o1.reshape(shp), o2.reshape(shp), om.reshape(shp))
